# Initial kernel scaffold; baseline (speedup 1.0000x reference)
import jax
import jax.numpy as jnp
from jax.experimental import pallas as pl


def kernel(x, edge_indices, Wl1, Wr1, b1, Wl2, Wr2, b2):
    raise NotImplementedError("write your pallas kernel here")


# speedup vs baseline: 3.2176x; 3.2176x over previous
import functools

import jax
import jax.numpy as jnp
from jax import lax
from jax.experimental import pallas as pl
from jax.experimental.pallas import tpu as pltpu
from jax.experimental.pallas import tpu_sc as plsc

_NC = 2
_NS = 16
_LW = 128


def _segsum_body(nch, n, npad, d,
                 x_hbm, srcp_hbm, dstp_hbm, zrow_hbm,
                 part_hbm,
                 src_t, dst_t, rows_v, sem, acc_sh):
  c = lax.axis_index("c")
  s = lax.axis_index("s")
  wid = c * _NS + s

  zrows = npad // _NS

  pltpu.sync_copy(srcp_hbm.at[pl.ds(wid * nch, nch)], src_t)
  pltpu.sync_copy(dstp_hbm.at[pl.ds(wid * nch, nch)], dst_t)
  pltpu.sync_copy(zrow_hbm, acc_sh.at[pl.ds(s * zrows, zrows)])
  plsc.subcore_barrier()

  def chunk(j, carry):
    pltpu.async_copy(x_hbm.at[src_t.at[j]], rows_v, sem).wait()
    pltpu.sync_copy(rows_v, acc_sh.at[dst_t.at[j]], add=True)
    return carry

  lax.fori_loop(0, nch, chunk, 0)
  plsc.subcore_barrier()

  pltpu.sync_copy(acc_sh.at[pl.ds(s * zrows, zrows)],
                  part_hbm.at[c, pl.ds(s * zrows, zrows)])


def _count_body(nch, npad, d,
                dstp_hbm, zcnt_hbm, ones_hbm,
                cntp_hbm,
                dst_t, ones_v, cnt_sh):
  c = lax.axis_index("c")
  s = lax.axis_index("s")
  wid = c * _NS + s
  zrows = npad // _NS

  pltpu.sync_copy(dstp_hbm.at[pl.ds(wid * nch, nch)], dst_t)
  pltpu.sync_copy(ones_hbm, ones_v)
  pltpu.sync_copy(zcnt_hbm, cnt_sh.at[pl.ds(s * zrows, zrows)])
  plsc.subcore_barrier()

  def chunk(j, carry):
    pltpu.sync_copy(ones_v, cnt_sh.at[dst_t.at[j]], add=True)
    return carry

  lax.fori_loop(0, nch, chunk, 0)
  plsc.subcore_barrier()
  pltpu.sync_copy(cnt_sh.at[pl.ds(s * zrows, zrows)],
                  cntp_hbm.at[c, pl.ds(s * zrows, zrows)])


@functools.partial(jax.jit, static_argnums=(1, 2))
def _sc_counts(dstp, n, d):
  nt = _NC * _NS
  nch = dstp.shape[0] // nt
  npad = (n + 128) // 128 * 128
  zcnt = jnp.zeros((npad // _NS, d), jnp.float32)
  ones = jnp.ones((_LW, d), jnp.float32)
  mesh = plsc.VectorSubcoreMesh(core_axis_name="c", subcore_axis_name="s")
  fn = pl.kernel(
      functools.partial(_count_body, nch, npad, d),
      out_type=jax.ShapeDtypeStruct((_NC, npad, d), jnp.float32),
      mesh=mesh,
      scratch_types=[
          pltpu.VMEM((nch, _LW), jnp.int32),
          pltpu.VMEM((_LW, d), jnp.float32),
          pltpu.VMEM_SHARED((npad, d), jnp.float32),
      ],
  )
  return fn(dstp, zcnt, ones)


@functools.partial(jax.jit, static_argnums=(3, 4))
def _sc_segsum(x, srcp, dstp, n, d):
  nt = _NC * _NS
  nch = srcp.shape[0] // nt
  npad = (n + 128) // 128 * 128

  zrow = jnp.zeros((npad // _NS, d), jnp.float32)

  mesh = plsc.VectorSubcoreMesh(core_axis_name="c", subcore_axis_name="s")
  fn = pl.kernel(
      functools.partial(_segsum_body, nch, n, npad, d),
      out_type=jax.ShapeDtypeStruct((_NC, npad, d), jnp.float32),
      mesh=mesh,
      scratch_types=[
          pltpu.VMEM((nch, _LW), jnp.int32),
          pltpu.VMEM((nch, _LW), jnp.int32),
          pltpu.VMEM((_LW, d), jnp.float32),
          pltpu.SemaphoreType.DMA,
          pltpu.VMEM_SHARED((npad, d), jnp.float32),
      ],
  )
  return fn(x, srcp, dstp, zrow)


def _phase_body(final, p_ref, cnt_ref, x_ref, wl_ref, wr_ref, b_ref, o_ref):
  s = p_ref[0] + p_ref[1]
  cnt = cnt_ref[0][:, :1] + cnt_ref[1][:, :1]
  mean = s / jnp.maximum(cnt, 1.0)
  h = (jnp.dot(mean, wl_ref[...], preferred_element_type=jnp.float32)
       + jnp.dot(x_ref[...], wr_ref[...], preferred_element_type=jnp.float32)
       + b_ref[...])
  if final:
    nrm = jnp.sqrt(jnp.sum(h * h, axis=1, keepdims=True))
    o_ref[...] = h / jnp.maximum(nrm, 1e-12)
  else:
    o_ref[...] = jnp.where(h >= 0, h, 0.01 * h)


@functools.partial(jax.jit, static_argnums=(6,))
def _tc_phase(part, cntp, x, wlt, wrt, b, final):
  n, d = x.shape
  r = 1000
  grid = (n // r,)
  return pl.pallas_call(
      functools.partial(_phase_body, final),
      grid=grid,
      in_specs=[
          pl.BlockSpec((_NC, r, d), lambda i: (0, i, 0)),
          pl.BlockSpec((_NC, r, d), lambda i: (0, i, 0)),
          pl.BlockSpec((r, d), lambda i: (i, 0)),
          pl.BlockSpec((d, d), lambda i: (0, 0)),
          pl.BlockSpec((d, d), lambda i: (0, 0)),
          pl.BlockSpec((1, d), lambda i: (0, 0)),
      ],
      out_specs=pl.BlockSpec((r, d), lambda i: (i, 0)),
      out_shape=jax.ShapeDtypeStruct((n, d), jnp.float32),
  )(part, cntp, x, wlt, wrt, b)


def kernel(x, edge_indices, Wl1, Wr1, b1, Wl2, Wr2, b2):
  n, d = x.shape
  e = edge_indices.shape[1]
  nt = _NC * _NS
  nch = -(-e // (_LW * nt))
  nch = (nch + 7) // 8 * 8
  epad = nch * _LW * nt

  src = jnp.concatenate(
      [edge_indices[0], jnp.zeros((epad - e,), jnp.int32)]).reshape(-1, _LW)
  dst = jnp.concatenate(
      [edge_indices[1], jnp.full((epad - e,), n, jnp.int32)]).reshape(-1, _LW)

  cp = _sc_counts(dst, n, d)
  p1 = _sc_segsum(x, src, dst, n, d)
  h = _tc_phase(p1, cp, x, Wl1.T, Wr1.T, b1.reshape(1, d), False)
  p2 = _sc_segsum(h, src, dst, n, d)
  return _tc_phase(p2, cp, h, Wl2.T, Wr2.T, b2.reshape(1, d), True)


# speedup vs baseline: 3.4228x; 1.0638x over previous
import functools

import jax
import jax.numpy as jnp
from jax import lax
from jax.experimental import pallas as pl
from jax.experimental.pallas import tpu as pltpu
from jax.experimental.pallas import tpu_sc as plsc

_NC = 2
_NS = 16
_LW = 128


def _segsum_body(nch, n, npad, d,
                 x_hbm, srcp_hbm, dstp_hbm, zrow_hbm,
                 part_hbm,
                 src_t, dst_t, rows_a, rows_b, sem_a, sem_b, acc_sh):
  c = lax.axis_index("c")
  s = lax.axis_index("s")
  wid = c * _NS + s

  zrows = npad // _NS
  nh = nch // 2

  pltpu.sync_copy(zrow_hbm, acc_sh.at[pl.ds(s * zrows, zrows)])
  plsc.subcore_barrier()

  def pair(t, carry):
    ja = 2 * t
    jb = 2 * t + 1
    pltpu.make_async_copy(x_hbm.at[src_t.at[ja]], rows_a, sem_a).wait()
    pltpu.async_copy(x_hbm.at[src_t.at[jb]], rows_b, sem_b)
    pltpu.sync_copy(rows_a, acc_sh.at[dst_t.at[ja]], add=True)
    pltpu.make_async_copy(x_hbm.at[src_t.at[jb]], rows_b, sem_b).wait()

    @pl.when(jb + 1 < nh)
    def _():
      pltpu.async_copy(x_hbm.at[src_t.at[jb + 1]], rows_a, sem_a)

    pltpu.sync_copy(rows_b, acc_sh.at[dst_t.at[jb]], add=True)
    return carry

  for h in range(2):
    pltpu.sync_copy(srcp_hbm.at[pl.ds(wid * nch + h * nh, nh)], src_t)
    pltpu.sync_copy(dstp_hbm.at[pl.ds(wid * nch + h * nh, nh)], dst_t)
    pltpu.async_copy(x_hbm.at[src_t.at[0]], rows_a, sem_a)
    lax.fori_loop(0, nh // 2, pair, 0)

  plsc.subcore_barrier()

  pltpu.sync_copy(acc_sh.at[pl.ds(s * zrows, zrows)],
                  part_hbm.at[c, pl.ds(s * zrows, zrows)])


def _count_body(nch, npad, d,
                dstp_hbm, zcnt_hbm, ones_hbm,
                cntp_hbm,
                dst_t, ones_v, cnt_sh):
  c = lax.axis_index("c")
  s = lax.axis_index("s")
  wid = c * _NS + s
  zrows = npad // _NS

  pltpu.sync_copy(dstp_hbm.at[pl.ds(wid * nch, nch)], dst_t)
  pltpu.sync_copy(ones_hbm, ones_v)
  pltpu.sync_copy(zcnt_hbm, cnt_sh.at[pl.ds(s * zrows, zrows)])
  plsc.subcore_barrier()

  def chunk(j, carry):
    pltpu.sync_copy(ones_v, cnt_sh.at[dst_t.at[j]], add=True)
    return carry

  lax.fori_loop(0, nch, chunk, 0)
  plsc.subcore_barrier()
  pltpu.sync_copy(cnt_sh.at[pl.ds(s * zrows, zrows)],
                  cntp_hbm.at[c, pl.ds(s * zrows, zrows)])


@functools.partial(jax.jit, static_argnums=(1, 2))
def _sc_counts(dstp, n, d):
  nt = _NC * _NS
  nch = dstp.shape[0] // nt
  npad = (n + 128) // 128 * 128
  zcnt = jnp.zeros((npad // _NS, d), jnp.float32)
  ones = jnp.ones((_LW, d), jnp.float32)
  mesh = plsc.VectorSubcoreMesh(core_axis_name="c", subcore_axis_name="s")
  fn = pl.kernel(
      functools.partial(_count_body, nch, npad, d),
      out_type=jax.ShapeDtypeStruct((_NC, npad, d), jnp.float32),
      mesh=mesh,
      scratch_types=[
          pltpu.VMEM((nch, _LW), jnp.int32),
          pltpu.VMEM((_LW, d), jnp.float32),
          pltpu.VMEM_SHARED((npad, d), jnp.float32),
      ],
  )
  return fn(dstp, zcnt, ones)


@functools.partial(jax.jit, static_argnums=(3, 4))
def _sc_segsum(x, srcp, dstp, n, d):
  nt = _NC * _NS
  nch = srcp.shape[0] // nt
  npad = (n + 128) // 128 * 128

  zrow = jnp.zeros((npad // _NS, d), jnp.float32)

  mesh = plsc.VectorSubcoreMesh(core_axis_name="c", subcore_axis_name="s")
  fn = pl.kernel(
      functools.partial(_segsum_body, nch, n, npad, d),
      out_type=jax.ShapeDtypeStruct((_NC, npad, d), jnp.float32),
      mesh=mesh,
      scratch_types=[
          pltpu.VMEM((nch // 2, _LW), jnp.int32),
          pltpu.VMEM((nch // 2, _LW), jnp.int32),
          pltpu.VMEM((_LW, d), jnp.float32),
          pltpu.VMEM((_LW, d), jnp.float32),
          pltpu.SemaphoreType.DMA,
          pltpu.SemaphoreType.DMA,
          pltpu.VMEM_SHARED((npad, d), jnp.float32),
      ],
  )
  return fn(x, srcp, dstp, zrow)


def _phase_body(final, p_ref, cnt_ref, x_ref, wl_ref, wr_ref, b_ref, o_ref):
  s = p_ref[0] + p_ref[1]
  cnt = cnt_ref[0][:, :1] + cnt_ref[1][:, :1]
  mean = s / jnp.maximum(cnt, 1.0)
  h = (jnp.dot(mean, wl_ref[...], preferred_element_type=jnp.float32)
       + jnp.dot(x_ref[...], wr_ref[...], preferred_element_type=jnp.float32)
       + b_ref[...])
  if final:
    nrm = jnp.sqrt(jnp.sum(h * h, axis=1, keepdims=True))
    o_ref[...] = h / jnp.maximum(nrm, 1e-12)
  else:
    o_ref[...] = jnp.where(h >= 0, h, 0.01 * h)


@functools.partial(jax.jit, static_argnums=(6,))
def _tc_phase(part, cntp, x, wlt, wrt, b, final):
  n, d = x.shape
  r = 1000
  grid = (n // r,)
  return pl.pallas_call(
      functools.partial(_phase_body, final),
      grid=grid,
      in_specs=[
          pl.BlockSpec((_NC, r, d), lambda i: (0, i, 0)),
          pl.BlockSpec((_NC, r, d), lambda i: (0, i, 0)),
          pl.BlockSpec((r, d), lambda i: (i, 0)),
          pl.BlockSpec((d, d), lambda i: (0, 0)),
          pl.BlockSpec((d, d), lambda i: (0, 0)),
          pl.BlockSpec((1, d), lambda i: (0, 0)),
      ],
      out_specs=pl.BlockSpec((r, d), lambda i: (i, 0)),
      out_shape=jax.ShapeDtypeStruct((n, d), jnp.float32),
  )(part, cntp, x, wlt, wrt, b)


def kernel(x, edge_indices, Wl1, Wr1, b1, Wl2, Wr2, b2):
  n, d = x.shape
  e = edge_indices.shape[1]
  nt = _NC * _NS
  nch = -(-e // (_LW * nt))
  nch = (nch + 7) // 8 * 8
  epad = nch * _LW * nt

  src = jnp.concatenate(
      [edge_indices[0], jnp.zeros((epad - e,), jnp.int32)]).reshape(-1, _LW)
  dst = jnp.concatenate(
      [edge_indices[1], jnp.full((epad - e,), n, jnp.int32)]).reshape(-1, _LW)

  cp = _sc_counts(dst, n, d)
  p1 = _sc_segsum(x, src, dst, n, d)
  h = _tc_phase(p1, cp, x, Wl1.T, Wr1.T, b1.reshape(1, d), False)
  p2 = _sc_segsum(h, src, dst, n, d)
  return _tc_phase(p2, cp, h, Wl2.T, Wr2.T, b2.reshape(1, d), True)


# speedup vs baseline: 3.5321x; 1.0319x over previous
import functools

import jax
import jax.numpy as jnp
from jax import lax
from jax.experimental import pallas as pl
from jax.experimental.pallas import tpu as pltpu
from jax.experimental.pallas import tpu_sc as plsc

_NC = 2
_NS = 16
_LW = 128


def _segsum_body(nch, n, npad, d,
                 x_hbm, srcp_hbm, dstp_hbm, zrow_hbm,
                 part_hbm,
                 src_t, dst_t, rows_a, rows_b, sem_a, sem_b, acc_sh):
  c = lax.axis_index("c")
  s = lax.axis_index("s")
  wid = c * _NS + s

  zrows = npad // _NS
  nh = nch // 2

  pltpu.sync_copy(zrow_hbm, acc_sh.at[pl.ds(s * zrows, zrows)])
  plsc.subcore_barrier()

  def pair(t, carry):
    ja = 2 * t
    jb = 2 * t + 1
    pltpu.make_async_copy(x_hbm.at[src_t.at[ja]], rows_a, sem_a).wait()
    pltpu.sync_copy(rows_a, acc_sh.at[dst_t.at[ja]], add=True)

    @pl.when(ja + 2 < nh)
    def _():
      pltpu.async_copy(x_hbm.at[src_t.at[ja + 2]], rows_a, sem_a)

    pltpu.make_async_copy(x_hbm.at[src_t.at[jb]], rows_b, sem_b).wait()
    pltpu.sync_copy(rows_b, acc_sh.at[dst_t.at[jb]], add=True)

    @pl.when(jb + 2 < nh)
    def _():
      pltpu.async_copy(x_hbm.at[src_t.at[jb + 2]], rows_b, sem_b)

    return carry

  for h in range(2):
    pltpu.sync_copy(srcp_hbm.at[pl.ds(wid * nch + h * nh, nh)], src_t)
    pltpu.sync_copy(dstp_hbm.at[pl.ds(wid * nch + h * nh, nh)], dst_t)
    pltpu.async_copy(x_hbm.at[src_t.at[0]], rows_a, sem_a)
    pltpu.async_copy(x_hbm.at[src_t.at[1]], rows_b, sem_b)
    lax.fori_loop(0, nh // 2, pair, 0)

  plsc.subcore_barrier()

  pltpu.sync_copy(acc_sh.at[pl.ds(s * zrows, zrows)],
                  part_hbm.at[c, pl.ds(s * zrows, zrows)])


def _count_body(nch, npad, d,
                dstp_hbm, zcnt_hbm, ones_hbm,
                cntp_hbm,
                dst_t, ones_v, cnt_sh):
  c = lax.axis_index("c")
  s = lax.axis_index("s")
  wid = c * _NS + s
  zrows = npad // _NS

  pltpu.sync_copy(dstp_hbm.at[pl.ds(wid * nch, nch)], dst_t)
  pltpu.sync_copy(ones_hbm, ones_v)
  pltpu.sync_copy(zcnt_hbm, cnt_sh.at[pl.ds(s * zrows, zrows)])
  plsc.subcore_barrier()

  def chunk(j, carry):
    pltpu.sync_copy(ones_v, cnt_sh.at[dst_t.at[j]], add=True)
    return carry

  lax.fori_loop(0, nch, chunk, 0)
  plsc.subcore_barrier()
  pltpu.sync_copy(cnt_sh.at[pl.ds(s * zrows, zrows)],
                  cntp_hbm.at[c, pl.ds(s * zrows, zrows)])


@functools.partial(jax.jit, static_argnums=(1, 2))
def _sc_counts(dstp, n, d):
  nt = _NC * _NS
  nch = dstp.shape[0] // nt
  npad = (n + 128) // 128 * 128
  zcnt = jnp.zeros((npad // _NS, d), jnp.float32)
  ones = jnp.ones((_LW, d), jnp.float32)
  mesh = plsc.VectorSubcoreMesh(core_axis_name="c", subcore_axis_name="s")
  fn = pl.kernel(
      functools.partial(_count_body, nch, npad, d),
      out_type=jax.ShapeDtypeStruct((_NC, npad, d), jnp.float32),
      mesh=mesh,
      scratch_types=[
          pltpu.VMEM((nch, _LW), jnp.int32),
          pltpu.VMEM((_LW, d), jnp.float32),
          pltpu.VMEM_SHARED((npad, d), jnp.float32),
      ],
  )
  return fn(dstp, zcnt, ones)


@functools.partial(jax.jit, static_argnums=(3, 4))
def _sc_segsum(x, srcp, dstp, n, d):
  nt = _NC * _NS
  nch = srcp.shape[0] // nt
  npad = (n + 128) // 128 * 128

  zrow = jnp.zeros((npad // _NS, d), jnp.float32)

  mesh = plsc.VectorSubcoreMesh(core_axis_name="c", subcore_axis_name="s")
  fn = pl.kernel(
      functools.partial(_segsum_body, nch, n, npad, d),
      out_type=jax.ShapeDtypeStruct((_NC, npad, d), jnp.float32),
      mesh=mesh,
      scratch_types=[
          pltpu.VMEM((nch // 2, _LW), jnp.int32),
          pltpu.VMEM((nch // 2, _LW), jnp.int32),
          pltpu.VMEM((_LW, d), jnp.float32),
          pltpu.VMEM((_LW, d), jnp.float32),
          pltpu.SemaphoreType.DMA,
          pltpu.SemaphoreType.DMA,
          pltpu.VMEM_SHARED((npad, d), jnp.float32),
      ],
  )
  return fn(x, srcp, dstp, zrow)


def _phase_body(final, p_ref, cnt_ref, x_ref, wl_ref, wr_ref, b_ref, o_ref):
  s = p_ref[0] + p_ref[1]
  cnt = cnt_ref[0][:, :1] + cnt_ref[1][:, :1]
  mean = s / jnp.maximum(cnt, 1.0)
  h = (jnp.dot(mean, wl_ref[...], preferred_element_type=jnp.float32)
       + jnp.dot(x_ref[...], wr_ref[...], preferred_element_type=jnp.float32)
       + b_ref[...])
  if final:
    nrm = jnp.sqrt(jnp.sum(h * h, axis=1, keepdims=True))
    o_ref[...] = h / jnp.maximum(nrm, 1e-12)
  else:
    o_ref[...] = jnp.where(h >= 0, h, 0.01 * h)


@functools.partial(jax.jit, static_argnums=(6,))
def _tc_phase(part, cntp, x, wlt, wrt, b, final):
  n, d = x.shape
  r = 1000
  grid = (n // r,)
  return pl.pallas_call(
      functools.partial(_phase_body, final),
      grid=grid,
      in_specs=[
          pl.BlockSpec((_NC, r, d), lambda i: (0, i, 0)),
          pl.BlockSpec((_NC, r, d), lambda i: (0, i, 0)),
          pl.BlockSpec((r, d), lambda i: (i, 0)),
          pl.BlockSpec((d, d), lambda i: (0, 0)),
          pl.BlockSpec((d, d), lambda i: (0, 0)),
          pl.BlockSpec((1, d), lambda i: (0, 0)),
      ],
      out_specs=pl.BlockSpec((r, d), lambda i: (i, 0)),
      out_shape=jax.ShapeDtypeStruct((n, d), jnp.float32),
  )(part, cntp, x, wlt, wrt, b)


def kernel(x, edge_indices, Wl1, Wr1, b1, Wl2, Wr2, b2):
  n, d = x.shape
  e = edge_indices.shape[1]
  nt = _NC * _NS
  nch = -(-e // (_LW * nt))
  nch = (nch + 7) // 8 * 8
  epad = nch * _LW * nt

  src = jnp.concatenate(
      [edge_indices[0], jnp.zeros((epad - e,), jnp.int32)]).reshape(-1, _LW)
  dst = jnp.concatenate(
      [edge_indices[1], jnp.full((epad - e,), n, jnp.int32)]).reshape(-1, _LW)

  cp = _sc_counts(dst, n, d)
  p1 = _sc_segsum(x, src, dst, n, d)
  h = _tc_phase(p1, cp, x, Wl1.T, Wr1.T, b1.reshape(1, d), False)
  p2 = _sc_segsum(h, src, dst, n, d)
  return _tc_phase(p2, cp, h, Wl2.T, Wr2.T, b2.reshape(1, d), True)
